# baseline (speedup 1.0000x reference)
import functools

import jax
import jax.numpy as jnp
from jax.experimental import pallas as pl
from jax.experimental.pallas import tpu as pltpu

DIM = 1024
HIDDEN = 2048
E = 8
TOP_K = 2

TM = 512
HB = 1024
NH = HIDDEN // HB


def _dense_body(x_ref, wrt_ref, wgt_ref, wut_ref, wdt_ref, out_ref):
    t = pl.program_id(0)
    e = pl.program_id(1)
    h = pl.program_id(2)
    first = jnp.logical_and(e == 0, h == 0)
    last = jnp.logical_and(e == E - 1, h == NH - 1)

    x = x_ref[...]
    xb = x.astype(jnp.bfloat16)

    logits = jnp.dot(xb, wrt_ref[...].astype(jnp.bfloat16),
                     preferred_element_type=jnp.float32)
    lanes = jax.lax.broadcasted_iota(jnp.int32, logits.shape, 1)
    m1 = jnp.max(logits, axis=1, keepdims=True)
    i1 = jnp.min(jnp.where(logits == m1, lanes, E), axis=1, keepdims=True)
    l2 = jnp.where(lanes == i1, -1e30, logits)
    m2 = jnp.max(l2, axis=1, keepdims=True)
    i2 = jnp.min(jnp.where(l2 == m2, lanes, E), axis=1, keepdims=True)
    r = jnp.exp(m2 - m1)
    denom = 1.0 + r
    w_full = jnp.where(lanes == i1, 1.0 / denom,
                       jnp.where(lanes == i2, r / denom, 0.0))
    w_col = jnp.sum(jnp.where(lanes == e, w_full, 0.0), axis=1, keepdims=True)

    g = jnp.dot(xb, wgt_ref[0], preferred_element_type=jnp.float32)
    u = jnp.dot(xb, wut_ref[0], preferred_element_type=jnp.float32)
    act = (g * jax.nn.sigmoid(g)) * u
    part = jnp.dot(act.astype(jnp.bfloat16), wdt_ref[0],
                   preferred_element_type=jnp.float32)
    contrib = w_col * part

    @pl.when(first)
    def _():
        out_ref[...] = contrib

    @pl.when(jnp.logical_not(first))
    def _():
        out_ref[...] = out_ref[...] + contrib


@jax.jit
def kernel(hidden_states, Wg, Wu, Wd, Wr):
    b, s, d = hidden_states.shape
    T = b * s
    x = hidden_states.reshape(T, d)
    wrt = Wr.T
    wgt = Wg.transpose(0, 2, 1).astype(jnp.bfloat16)
    wut = Wu.transpose(0, 2, 1).astype(jnp.bfloat16)
    wdt = Wd.transpose(0, 2, 1).astype(jnp.bfloat16)

    nt = T // TM
    out = pl.pallas_call(
        _dense_body,
        grid=(nt, E, NH),
        in_specs=[
            pl.BlockSpec((TM, d), lambda t, e, h: (t, 0)),
            pl.BlockSpec((d, E), lambda t, e, h: (0, 0)),
            pl.BlockSpec((1, d, HB), lambda t, e, h: (e, 0, h)),
            pl.BlockSpec((1, d, HB), lambda t, e, h: (e, 0, h)),
            pl.BlockSpec((1, HB, d), lambda t, e, h: (e, h, 0)),
        ],
        out_specs=pl.BlockSpec((TM, d), lambda t, e, h: (t, 0)),
        out_shape=jax.ShapeDtypeStruct((T, d), jnp.float32),
    )(x, wrt, wgt, wut, wdt)
    return out.reshape(b, s, d)


# speedup vs baseline: 1.3139x; 1.3139x over previous
import functools

import jax
import jax.numpy as jnp
from jax import lax
from jax.experimental import pallas as pl
from jax.experimental.pallas import tpu as pltpu
from jax.experimental.pallas import tpu_sc as plsc

DIM = 1024
HIDDEN = 2048
E = 8
T = 2048

TM3 = 256
NT3 = 24
M_PAD = NT3 * TM3
CS = 256
NTP = 32

_NC, _NS = 2, 16
_NW = _NC * _NS


def _router_body(x_ref, wrt_ref, pos0_ref, pos1_ref, w0_ref, w1_ref,
                 te_ref, tv_ref):
    x = x_ref[...]
    xb = x.astype(jnp.bfloat16)
    logits = jnp.dot(xb, wrt_ref[...].astype(jnp.bfloat16),
                     preferred_element_type=jnp.float32)
    lanes = lax.broadcasted_iota(jnp.int32, (T, E), 1)
    m1 = jnp.max(logits, axis=1, keepdims=True)
    i1 = jnp.min(jnp.where(logits == m1, lanes, E), axis=1, keepdims=True)
    l2 = jnp.where(lanes == i1, -1e30, logits)
    m2 = jnp.max(l2, axis=1, keepdims=True)
    i2 = jnp.min(jnp.where(l2 == m2, lanes, E), axis=1, keepdims=True)
    r = jnp.exp(m2 - m1)
    den = 1.0 + r
    w0_ref[...] = 1.0 / den
    w1_ref[...] = r / den

    oh0 = (lanes == i1).astype(jnp.float32)
    oh1 = (lanes == i2).astype(jnp.float32)

    ii = lax.broadcasted_iota(jnp.int32, (CS, CS), 0)
    jj = lax.broadcasted_iota(jnp.int32, (CS, CS), 1)
    ltri = (jj < ii).astype(jnp.float32)
    off = jnp.zeros((1, E), jnp.float32)
    prefixes = []
    for oh in (oh0, oh1):
        parts = []
        for c in range(T // CS):
            blk = lax.slice(oh, (c * CS, 0), ((c + 1) * CS, E))
            ex = lax.dot(ltri, blk,
                         precision=lax.Precision.HIGHEST) + off
            off = off + jnp.sum(blk, axis=0, keepdims=True)
            parts.append(ex)
        prefixes.append(jnp.concatenate(parts, axis=0))
    counts = off
    padded = jnp.ceil(counts / TM3) * TM3

    acc = jnp.zeros((1, 1), jnp.float32)
    offp_parts = []
    for e in range(E):
        offp_parts.append(acc)
        acc = acc + lax.slice(padded, (0, e), (1, e + 1))
    offp = jnp.concatenate(offp_parts, axis=1)

    pos0 = jnp.sum(oh0 * (prefixes[0] + offp), axis=1, keepdims=True)
    pos1 = jnp.sum(oh1 * (prefixes[1] + offp), axis=1, keepdims=True)
    pos0_ref[...] = pos0.astype(jnp.int32)
    pos1_ref[...] = pos1.astype(jnp.int32)

    row0 = lax.broadcasted_iota(jnp.int32, (1, NTP), 1).astype(jnp.float32) * TM3
    te = jnp.zeros((1, NTP), jnp.float32)
    for e in range(E):
        oe = lax.slice(offp, (0, e), (1, e + 1))
        pe = lax.slice(padded, (0, e), (1, e + 1))
        inside = jnp.logical_and(row0 >= oe, row0 < oe + pe)
        te = te + e * inside.astype(jnp.float32)
    total = acc
    valid = row0 < total
    erow = lax.broadcasted_iota(jnp.int32, (1, E), 1).astype(jnp.float32)
    laste = jnp.max(jnp.where(counts > 0, erow, 0.0), axis=1, keepdims=True)
    te = jnp.where(valid, te, laste)
    te_ref[...] = te.astype(jnp.int32)
    tv_ref[...] = valid.astype(jnp.int32)


def _router_sort(x, wrt):
    outs = pl.pallas_call(
        _router_body,
        out_shape=[
            jax.ShapeDtypeStruct((T, 1), jnp.int32),
            jax.ShapeDtypeStruct((T, 1), jnp.int32),
            jax.ShapeDtypeStruct((T, 1), jnp.float32),
            jax.ShapeDtypeStruct((T, 1), jnp.float32),
            jax.ShapeDtypeStruct((1, NTP), jnp.int32),
            jax.ShapeDtypeStruct((1, NTP), jnp.int32),
        ],
    )(x, wrt)
    return outs


def _dispatch_sc(x, pos_flat):
    ch = 64
    mesh = plsc.VectorSubcoreMesh(core_axis_name="c", subcore_axis_name="s")

    @functools.partial(
        pl.kernel, mesh=mesh,
        out_type=jax.ShapeDtypeStruct((M_PAD, DIM), jnp.float32),
        scratch_types=[
            pltpu.VMEM((ch,), jnp.int32),
            pltpu.VMEM((ch, DIM), jnp.float32),
            pltpu.SemaphoreType.DMA,
        ],
    )
    def k(x_hbm, pos_hbm, xs_hbm, idx_v, rows_v, sem):
        wid = lax.axis_index("s") * _NC + lax.axis_index("c")
        npairs = 2 * T // _NW
        base = wid * npairs
        for j in range(npairs // ch):
            b = base + j * ch
            tok = lax.rem(b, T)
            pltpu.sync_copy(pos_hbm.at[pl.ds(b, ch)], idx_v)
            pltpu.sync_copy(x_hbm.at[pl.ds(tok, ch)], rows_v)
            pltpu.async_copy(rows_v, xs_hbm.at[idx_v], sem).wait()

    return k(x, pos_flat)


def _gmm_body(te_r, tv_r, x_ref, wg_ref, wu_ref, wd_ref, out_ref):
    m = pl.program_id(0)

    @pl.when(tv_r[m] == 1)
    def _():
        xb = x_ref[...].astype(jnp.bfloat16)
        g = jnp.dot(xb, wg_ref[0], preferred_element_type=jnp.float32)
        u = jnp.dot(xb, wu_ref[0], preferred_element_type=jnp.float32)
        act = (g * jax.nn.sigmoid(g)) * u
        out_ref[...] = jnp.dot(act.astype(jnp.bfloat16), wd_ref[0],
                               preferred_element_type=jnp.float32)


def _gmm(xs, wgt, wut, wdt, te, tv):
    grid_spec = pltpu.PrefetchScalarGridSpec(
        num_scalar_prefetch=2,
        grid=(NT3,),
        in_specs=[
            pl.BlockSpec((TM3, DIM), lambda m, te_r, tv_r: (m, 0)),
            pl.BlockSpec((1, DIM, HIDDEN), lambda m, te_r, tv_r: (te_r[m], 0, 0)),
            pl.BlockSpec((1, DIM, HIDDEN), lambda m, te_r, tv_r: (te_r[m], 0, 0)),
            pl.BlockSpec((1, HIDDEN, DIM), lambda m, te_r, tv_r: (te_r[m], 0, 0)),
        ],
        out_specs=pl.BlockSpec((TM3, DIM), lambda m, te_r, tv_r: (m, 0)),
    )
    return pl.pallas_call(
        _gmm_body,
        grid_spec=grid_spec,
        out_shape=jax.ShapeDtypeStruct((M_PAD, DIM), jnp.float32),
    )(te, tv, xs, wgt, wut, wdt)


def _combine_sc(y, pos0, pos1, w0, w1):
    ct = 16
    mesh = plsc.VectorSubcoreMesh(core_axis_name="c", subcore_axis_name="s")

    @functools.partial(
        pl.kernel, mesh=mesh,
        out_type=jax.ShapeDtypeStruct((T, DIM), jnp.float32),
        scratch_types=[
            pltpu.VMEM((ct,), jnp.int32),
            pltpu.VMEM((ct,), jnp.int32),
            pltpu.VMEM((ct,), jnp.float32),
            pltpu.VMEM((ct,), jnp.float32),
            pltpu.VMEM((ct, DIM), jnp.float32),
            pltpu.VMEM((ct, DIM), jnp.float32),
            pltpu.VMEM((ct, DIM), jnp.float32),
            pltpu.SemaphoreType.DMA,
        ],
    )
    def k(y_hbm, p0_hbm, p1_hbm, w0_hbm, w1_hbm, out_hbm,
          i0v, i1v, w0v, w1v, r0v, r1v, ov, sem):
        wid = lax.axis_index("s") * _NC + lax.axis_index("c")
        ntok = T // _NW
        for j in range(ntok // ct):
            b = wid * ntok + j * ct
            pltpu.sync_copy(p0_hbm.at[pl.ds(b, ct)], i0v)
            pltpu.sync_copy(p1_hbm.at[pl.ds(b, ct)], i1v)
            pltpu.sync_copy(w0_hbm.at[pl.ds(b, ct)], w0v)
            pltpu.sync_copy(w1_hbm.at[pl.ds(b, ct)], w1v)
            pltpu.async_copy(y_hbm.at[i0v], r0v, sem).wait()
            pltpu.async_copy(y_hbm.at[i1v], r1v, sem).wait()
            w0a = w0v[...]
            w1a = w1v[...]
            wa = [w0a[tt] for tt in range(ct)]
            wb = [w1a[tt] for tt in range(ct)]

            def cbody(c, carry):
                for tt in range(ct):
                    sl = pl.ds(c * 16, 16)
                    ov[tt, sl] = r0v[tt, sl] * wa[tt] + r1v[tt, sl] * wb[tt]
                return carry

            lax.fori_loop(0, DIM // 16, cbody, 0)
            pltpu.sync_copy(ov, out_hbm.at[pl.ds(b, ct)])

    return k(y, pos0, pos1, w0, w1)


@jax.jit
def kernel(hidden_states, Wg, Wu, Wd, Wr):
    b, s, d = hidden_states.shape
    x = hidden_states.reshape(T, d)
    wrt = Wr.T
    wgt = Wg.transpose(0, 2, 1).astype(jnp.bfloat16)
    wut = Wu.transpose(0, 2, 1).astype(jnp.bfloat16)
    wdt = Wd.transpose(0, 2, 1).astype(jnp.bfloat16)

    pos0, pos1, w0, w1, te, tv = _router_sort(x, wrt)
    pos_flat = jnp.concatenate([pos0[:, 0], pos1[:, 0]], axis=0)
    xs = _dispatch_sc(x, pos_flat)
    y = _gmm(xs, wgt, wut, wdt, te[0], tv[0])
    out = _combine_sc(y, pos0[:, 0], pos1[:, 0], w0[:, 0], w1[:, 0])
    return out.reshape(b, s, d)


# speedup vs baseline: 2.1415x; 1.6298x over previous
import functools

import jax
import jax.numpy as jnp
from jax import lax
from jax.experimental import pallas as pl
from jax.experimental.pallas import tpu as pltpu
from jax.experimental.pallas import tpu_sc as plsc

DIM = 1024
HIDDEN = 2048
E = 8
T = 2048

TM3 = 256
NT3 = 24
M_PAD = NT3 * TM3
CS = 256
NTP = 32

_NC, _NS = 2, 16
_NW = _NC * _NS


def _router_body(x_ref, wrt_ref, pf_ref, wf_ref, te_ref, tv_ref):
    x = x_ref[...]
    xb = x.astype(jnp.bfloat16)
    logits = lax.dot_general(xb, wrt_ref[...].astype(jnp.bfloat16),
                             (((1,), (1,)), ((), ())),
                             preferred_element_type=jnp.float32)
    lanes = lax.broadcasted_iota(jnp.int32, (T, E), 1)
    m1 = jnp.max(logits, axis=1, keepdims=True)
    i1 = jnp.min(jnp.where(logits == m1, lanes, E), axis=1, keepdims=True)
    l2 = jnp.where(lanes == i1, -1e30, logits)
    m2 = jnp.max(l2, axis=1, keepdims=True)
    i2 = jnp.min(jnp.where(l2 == m2, lanes, E), axis=1, keepdims=True)
    r = jnp.exp(m2 - m1)
    den = 1.0 + r
    wf_ref[pl.ds(0, T), :] = 1.0 / den
    wf_ref[pl.ds(T, T), :] = r / den

    oh0 = (lanes == i1).astype(jnp.float32)
    oh1 = (lanes == i2).astype(jnp.float32)

    ii = lax.broadcasted_iota(jnp.int32, (CS, CS), 0)
    jj = lax.broadcasted_iota(jnp.int32, (CS, CS), 1)
    ltri = (jj < ii).astype(jnp.float32)
    off = jnp.zeros((1, E), jnp.float32)
    prefixes = []
    for oh in (oh0, oh1):
        parts = []
        for c in range(T // CS):
            blk = lax.slice(oh, (c * CS, 0), ((c + 1) * CS, E))
            ex = lax.dot(ltri, blk,
                         precision=lax.Precision.HIGHEST) + off
            off = off + jnp.sum(blk, axis=0, keepdims=True)
            parts.append(ex)
        prefixes.append(jnp.concatenate(parts, axis=0))
    counts = off
    padded = jnp.ceil(counts / TM3) * TM3

    acc = jnp.zeros((1, 1), jnp.float32)
    offp_parts = []
    for e in range(E):
        offp_parts.append(acc)
        acc = acc + lax.slice(padded, (0, e), (1, e + 1))
    offp = jnp.concatenate(offp_parts, axis=1)

    pos0 = jnp.sum(oh0 * (prefixes[0] + offp), axis=1, keepdims=True)
    pos1 = jnp.sum(oh1 * (prefixes[1] + offp), axis=1, keepdims=True)
    pf_ref[pl.ds(0, T), :] = pos0.astype(jnp.int32)
    pf_ref[pl.ds(T, T), :] = pos1.astype(jnp.int32)

    row0 = lax.broadcasted_iota(jnp.int32, (1, NTP), 1).astype(jnp.float32) * TM3
    te = jnp.zeros((1, NTP), jnp.float32)
    for e in range(E):
        oe = lax.slice(offp, (0, e), (1, e + 1))
        pe = lax.slice(padded, (0, e), (1, e + 1))
        inside = jnp.logical_and(row0 >= oe, row0 < oe + pe)
        te = te + e * inside.astype(jnp.float32)
    total = acc
    valid = row0 < total
    erow = lax.broadcasted_iota(jnp.int32, (1, E), 1).astype(jnp.float32)
    laste = jnp.max(jnp.where(counts > 0, erow, 0.0), axis=1, keepdims=True)
    te = jnp.where(valid, te, laste)
    te_ref[...] = te.astype(jnp.int32)
    tv_ref[...] = valid.astype(jnp.int32)


def _router_sort(x, wrt):
    outs = pl.pallas_call(
        _router_body,
        out_shape=[
            jax.ShapeDtypeStruct((2 * T, 1), jnp.int32),
            jax.ShapeDtypeStruct((2 * T, 1), jnp.float32),
            jax.ShapeDtypeStruct((1, NTP), jnp.int32),
            jax.ShapeDtypeStruct((1, NTP), jnp.int32),
        ],
    )(x, wrt)
    return outs


def _dispatch_sc(x, pos_flat):
    ch = 64
    mesh = plsc.VectorSubcoreMesh(core_axis_name="c", subcore_axis_name="s")

    @functools.partial(
        pl.kernel, mesh=mesh,
        out_type=jax.ShapeDtypeStruct((M_PAD, DIM), jnp.float32),
        scratch_types=[
            pltpu.VMEM((ch,), jnp.int32),
            pltpu.VMEM((ch, DIM), jnp.float32),
            pltpu.SemaphoreType.DMA,
        ],
    )
    def k(x_hbm, pos_hbm, xs_hbm, idx_v, rows_v, sem):
        wid = lax.axis_index("s") * _NC + lax.axis_index("c")
        npairs = 2 * T // _NW
        base = wid * npairs
        for j in range(npairs // ch):
            b = base + j * ch
            tok = lax.rem(b, T)
            pltpu.sync_copy(pos_hbm.at[pl.ds(b, ch)], idx_v)
            pltpu.sync_copy(x_hbm.at[pl.ds(tok, ch)], rows_v)
            pltpu.async_copy(rows_v, xs_hbm.at[idx_v], sem).wait()

    return k(x, pos_flat)


def _gmm_body(te_r, tv_r, x_ref, wg_ref, wu_ref, wd_ref, out_ref):
    m = pl.program_id(0)

    @pl.when(tv_r[m] == 1)
    def _():
        xb = x_ref[...].astype(jnp.bfloat16)
        wgb = wg_ref[0].astype(jnp.bfloat16)
        wub = wu_ref[0].astype(jnp.bfloat16)
        wdb = wd_ref[0].astype(jnp.bfloat16)
        nt = (((1,), (1,)), ((), ()))
        g = lax.dot_general(xb, wgb, nt, preferred_element_type=jnp.float32)
        u = lax.dot_general(xb, wub, nt, preferred_element_type=jnp.float32)
        act = (g * jax.nn.sigmoid(g)) * u
        out_ref[...] = lax.dot_general(act.astype(jnp.bfloat16), wdb, nt,
                                       preferred_element_type=jnp.float32)


def _gmm(xs, wgt, wut, wdt, te, tv):
    grid_spec = pltpu.PrefetchScalarGridSpec(
        num_scalar_prefetch=2,
        grid=(NT3,),
        in_specs=[
            pl.BlockSpec((TM3, DIM), lambda m, te_r, tv_r: (m, 0)),
            pl.BlockSpec((1, HIDDEN, DIM), lambda m, te_r, tv_r: (te_r[m], 0, 0)),
            pl.BlockSpec((1, HIDDEN, DIM), lambda m, te_r, tv_r: (te_r[m], 0, 0)),
            pl.BlockSpec((1, DIM, HIDDEN), lambda m, te_r, tv_r: (te_r[m], 0, 0)),
        ],
        out_specs=pl.BlockSpec((TM3, DIM), lambda m, te_r, tv_r: (m, 0)),
    )
    return pl.pallas_call(
        _gmm_body,
        grid_spec=grid_spec,
        out_shape=jax.ShapeDtypeStruct((M_PAD, DIM), jnp.float32),
    )(te, tv, xs, wgt, wut, wdt)


def _combine_sc(y, pos_flat, w_flat):
    ct = 16
    mesh = plsc.VectorSubcoreMesh(core_axis_name="c", subcore_axis_name="s")

    @functools.partial(
        pl.kernel, mesh=mesh,
        out_type=jax.ShapeDtypeStruct((T, DIM), jnp.float32),
        scratch_types=[
            pltpu.VMEM((ct,), jnp.int32),
            pltpu.VMEM((ct,), jnp.int32),
            pltpu.VMEM((ct,), jnp.float32),
            pltpu.VMEM((ct,), jnp.float32),
            pltpu.VMEM((ct, DIM), jnp.float32),
            pltpu.VMEM((ct, DIM), jnp.float32),
            pltpu.VMEM((ct, DIM), jnp.float32),
            pltpu.SemaphoreType.DMA,
            pltpu.SemaphoreType.DMA,
        ],
    )
    def k(y_hbm, pf_hbm, wf_hbm, out_hbm,
          i0v, i1v, w0v, w1v, r0v, r1v, ov, sem0, sem1):
        wid = lax.axis_index("s") * _NC + lax.axis_index("c")
        ntok = T // _NW
        for j in range(ntok // ct):
            b = wid * ntok + j * ct
            pltpu.sync_copy(pf_hbm.at[pl.ds(b, ct)], i0v)
            pltpu.sync_copy(pf_hbm.at[pl.ds(T + b, ct)], i1v)
            pltpu.sync_copy(wf_hbm.at[pl.ds(b, ct)], w0v)
            pltpu.sync_copy(wf_hbm.at[pl.ds(T + b, ct)], w1v)
            c0 = pltpu.async_copy(y_hbm.at[i0v], r0v, sem0)
            c1 = pltpu.async_copy(y_hbm.at[i1v], r1v, sem1)
            c0.wait()
            c1.wait()
            w0a = w0v[...]
            w1a = w1v[...]
            wa = [w0a[tt] for tt in range(ct)]
            wb = [w1a[tt] for tt in range(ct)]

            def cbody(c, carry):
                for tt in range(ct):
                    sl = pl.ds(c * 16, 16)
                    ov[tt, sl] = r0v[tt, sl] * wa[tt] + r1v[tt, sl] * wb[tt]
                return carry

            lax.fori_loop(0, DIM // 16, cbody, 0)
            pltpu.sync_copy(ov, out_hbm.at[pl.ds(b, ct)])

    return k(y, pos_flat, w_flat)


@jax.jit
def kernel(hidden_states, Wg, Wu, Wd, Wr):
    b, s, d = hidden_states.shape
    x = hidden_states.reshape(T, d)

    pf, wf, te, tv = _router_sort(x, Wr)
    pos_flat = pf.reshape(2 * T)
    w_flat = wf.reshape(2 * T)
    xs = _dispatch_sc(x, pos_flat)
    y = _gmm(xs, Wg, Wu, Wd, te[0], tv[0])
    out = _combine_sc(y, pos_flat, w_flat)
    return out.reshape(b, s, d)


# speedup vs baseline: 2.4553x; 1.1466x over previous
import functools

import jax
import jax.numpy as jnp
from jax import lax
from jax.experimental import pallas as pl
from jax.experimental.pallas import tpu as pltpu
from jax.experimental.pallas import tpu_sc as plsc

DIM = 1024
HIDDEN = 2048
E = 8
T = 2048

TM3 = 256
NT3 = 24
M_PAD = NT3 * TM3
CS = 256
NTP = 32

_NC, _NS = 2, 16
_NW = _NC * _NS


def _router_body(x_ref, wrt_ref, pf_ref, wf_ref, te_ref, tv_ref,
                 fs_ref, ne_ref, od_ref):
    x = x_ref[...]
    xb = x.astype(jnp.bfloat16)
    logits = lax.dot_general(xb, wrt_ref[...].astype(jnp.bfloat16),
                             (((1,), (1,)), ((), ())),
                             preferred_element_type=jnp.float32)
    lanes = lax.broadcasted_iota(jnp.int32, (T, E), 1)
    m1 = jnp.max(logits, axis=1, keepdims=True)
    i1 = jnp.min(jnp.where(logits == m1, lanes, E), axis=1, keepdims=True)
    l2 = jnp.where(lanes == i1, -1e30, logits)
    m2 = jnp.max(l2, axis=1, keepdims=True)
    i2 = jnp.min(jnp.where(l2 == m2, lanes, E), axis=1, keepdims=True)
    r = jnp.exp(m2 - m1)
    den = 1.0 + r
    wf_ref[pl.ds(0, T), :] = 1.0 / den
    wf_ref[pl.ds(T, T), :] = r / den

    oh0 = (lanes == i1).astype(jnp.float32)
    oh1 = (lanes == i2).astype(jnp.float32)

    ii = lax.broadcasted_iota(jnp.int32, (CS, CS), 0)
    jj = lax.broadcasted_iota(jnp.int32, (CS, CS), 1)
    ltri = (jj < ii).astype(jnp.float32)
    off = jnp.zeros((1, E), jnp.float32)
    prefixes = []
    for oh in (oh0, oh1):
        parts = []
        for c in range(T // CS):
            blk = lax.slice(oh, (c * CS, 0), ((c + 1) * CS, E))
            ex = lax.dot(ltri, blk,
                         precision=lax.Precision.HIGHEST) + off
            off = off + jnp.sum(blk, axis=0, keepdims=True)
            parts.append(ex)
        prefixes.append(jnp.concatenate(parts, axis=0))
    counts = off
    padded = jnp.ceil(counts / TM3) * TM3

    acc = jnp.zeros((1, 1), jnp.float32)
    offp_parts = []
    for e in range(E):
        offp_parts.append(acc)
        acc = acc + lax.slice(padded, (0, e), (1, e + 1))
    offp = jnp.concatenate(offp_parts, axis=1)

    pos0 = jnp.sum(oh0 * (prefixes[0] + offp), axis=1, keepdims=True)
    pos1 = jnp.sum(oh1 * (prefixes[1] + offp), axis=1, keepdims=True)
    pf_ref[pl.ds(0, T), :] = pos0.astype(jnp.int32)
    pf_ref[pl.ds(T, T), :] = pos1.astype(jnp.int32)

    row0 = lax.broadcasted_iota(jnp.int32, (1, NTP), 1).astype(jnp.float32) * TM3
    active = (counts > 0)
    nxt_after = jnp.full((1, 1), -1.0, jnp.float32)
    nxts = [None] * E
    for e in range(E - 1, -1, -1):
        nxts[e] = nxt_after
        ae = lax.slice(active, (0, e), (1, e + 1))
        nxt_after = jnp.where(ae, float(e), nxt_after)
    nordacc = jnp.zeros((1, 1), jnp.float32)
    te = jnp.zeros((1, NTP), jnp.float32)
    fs = jnp.zeros((1, NTP), jnp.float32)
    ne = jnp.zeros((1, NTP), jnp.float32)
    od = jnp.zeros((1, NTP), jnp.float32)
    for e in range(E):
        oe = lax.slice(offp, (0, e), (1, e + 1))
        pe = lax.slice(padded, (0, e), (1, e + 1))
        inside = jnp.logical_and(row0 >= oe, row0 < oe + pe).astype(jnp.float32)
        te = te + e * inside
        fs = fs + jnp.where(row0 == oe, inside, 0.0)
        nv = jnp.where(nxts[e] < 0, float(e), nxts[e])
        ne = ne + nv * inside
        od = od + nordacc * inside
        ae = lax.slice(active, (0, e), (1, e + 1))
        nordacc = nordacc + ae.astype(jnp.float32)
    total = acc
    valid = row0 < total
    erow = lax.broadcasted_iota(jnp.int32, (1, E), 1).astype(jnp.float32)
    laste = jnp.max(jnp.where(counts > 0, erow, 0.0), axis=1, keepdims=True)
    te = jnp.where(valid, te, laste)
    te_ref[...] = te.astype(jnp.int32)
    tv_ref[...] = valid.astype(jnp.int32)
    fs_ref[...] = fs.astype(jnp.int32)
    ne_ref[...] = ne.astype(jnp.int32)
    od_ref[...] = (od - 2.0 * jnp.floor(od * 0.5)).astype(jnp.int32)


def _router_sort(x, wrt):
    outs = pl.pallas_call(
        _router_body,
        out_shape=[
            jax.ShapeDtypeStruct((2 * T, 1), jnp.int32),
            jax.ShapeDtypeStruct((2 * T, 1), jnp.float32),
            jax.ShapeDtypeStruct((1, NTP), jnp.int32),
            jax.ShapeDtypeStruct((1, NTP), jnp.int32),
            jax.ShapeDtypeStruct((1, NTP), jnp.int32),
            jax.ShapeDtypeStruct((1, NTP), jnp.int32),
            jax.ShapeDtypeStruct((1, NTP), jnp.int32),
        ],
    )(x, wrt)
    return outs


def _dispatch_sc(x, pos_flat):
    ch = 64
    mesh = plsc.VectorSubcoreMesh(core_axis_name="c", subcore_axis_name="s")

    @functools.partial(
        pl.kernel, mesh=mesh,
        out_type=jax.ShapeDtypeStruct((M_PAD, DIM), jnp.float32),
        scratch_types=[
            pltpu.VMEM((ch,), jnp.int32),
            pltpu.VMEM((ch, DIM), jnp.float32),
            pltpu.SemaphoreType.DMA,
        ],
    )
    def k(x_hbm, pos_hbm, xs_hbm, idx_v, rows_v, sem):
        wid = lax.axis_index("s") * _NC + lax.axis_index("c")
        npairs = 2 * T // _NW
        base = wid * npairs
        for j in range(npairs // ch):
            b = base + j * ch
            tok = lax.rem(b, T)
            pltpu.sync_copy(pos_hbm.at[pl.ds(b, ch)], idx_v)
            pltpu.sync_copy(x_hbm.at[pl.ds(tok, ch)], rows_v)
            pltpu.async_copy(rows_v, xs_hbm.at[idx_v], sem).wait()

    return k(x, pos_flat)


def _gmm_body(te_r, tv_r, fs_r, ne_r, od_r,
              x_ref, wg_hbm, wu_hbm, wd_hbm, out_ref,
              wg0, wu0, wd0, wg1, wu1, wd1,
              sg0, su0, sd0, sg1, su1, sd1):
    m = pl.program_id(0)
    e = te_r[m]
    par = od_r[m]
    nxt = ne_r[m]
    first = fs_r[m] == 1

    def _issue(eidx, bg, bu, bd, s0, s1, s2):
        pltpu.make_async_copy(wg_hbm.at[eidx], bg, s0).start()
        pltpu.make_async_copy(wu_hbm.at[eidx], bu, s1).start()
        pltpu.make_async_copy(wd_hbm.at[eidx], bd, s2).start()

    def _wait(eidx, bg, bu, bd, s0, s1, s2):
        pltpu.make_async_copy(wg_hbm.at[eidx], bg, s0).wait()
        pltpu.make_async_copy(wu_hbm.at[eidx], bu, s1).wait()
        pltpu.make_async_copy(wd_hbm.at[eidx], bd, s2).wait()

    @pl.when(m == 0)
    def _():
        _issue(e, wg0, wu0, wd0, sg0, su0, sd0)

    @pl.when(jnp.logical_and(first, nxt != e))
    def _():
        @pl.when(par == 0)
        def _():
            _issue(nxt, wg1, wu1, wd1, sg1, su1, sd1)

        @pl.when(par == 1)
        def _():
            _issue(nxt, wg0, wu0, wd0, sg0, su0, sd0)

    @pl.when(first)
    def _():
        @pl.when(par == 0)
        def _():
            _wait(e, wg0, wu0, wd0, sg0, su0, sd0)

        @pl.when(par == 1)
        def _():
            _wait(e, wg1, wu1, wd1, sg1, su1, sd1)

    def _compute(bg, bu, bd):
        xv = x_ref[...]
        nt = (((1,), (1,)), ((), ()))
        g = lax.dot_general(xv, bg[...], nt, preferred_element_type=jnp.float32)
        u = lax.dot_general(xv, bu[...], nt, preferred_element_type=jnp.float32)
        act = (g * jax.nn.sigmoid(g)) * u
        out_ref[...] = lax.dot_general(act, bd[...], nt,
                                       preferred_element_type=jnp.float32)

    @pl.when(jnp.logical_and(tv_r[m] == 1, par == 0))
    def _():
        _compute(wg0, wu0, wd0)

    @pl.when(jnp.logical_and(tv_r[m] == 1, par == 1))
    def _():
        _compute(wg1, wu1, wd1)


def _gmm(xs, wgt, wut, wdt, te, tv, fs, ne, od):
    grid_spec = pltpu.PrefetchScalarGridSpec(
        num_scalar_prefetch=5,
        grid=(NT3,),
        in_specs=[
            pl.BlockSpec((TM3, DIM), lambda m, *_: (m, 0)),
            pl.BlockSpec(memory_space=pltpu.MemorySpace.HBM),
            pl.BlockSpec(memory_space=pltpu.MemorySpace.HBM),
            pl.BlockSpec(memory_space=pltpu.MemorySpace.HBM),
        ],
        out_specs=pl.BlockSpec((TM3, DIM), lambda m, *_: (m, 0)),
        scratch_shapes=[
            pltpu.VMEM((HIDDEN, DIM), jnp.float32),
            pltpu.VMEM((HIDDEN, DIM), jnp.float32),
            pltpu.VMEM((DIM, HIDDEN), jnp.float32),
            pltpu.VMEM((HIDDEN, DIM), jnp.float32),
            pltpu.VMEM((HIDDEN, DIM), jnp.float32),
            pltpu.VMEM((DIM, HIDDEN), jnp.float32),
        ] + [pltpu.SemaphoreType.DMA] * 6,
    )
    return pl.pallas_call(
        _gmm_body,
        grid_spec=grid_spec,
        out_shape=jax.ShapeDtypeStruct((M_PAD, DIM), jnp.float32),
        compiler_params=pltpu.CompilerParams(vmem_limit_bytes=100 * 1024 * 1024),
    )(te, tv, fs, ne, od, xs, wgt, wut, wdt)


def _combine_sc(y, pos_flat, w_flat):
    ct = 16
    mesh = plsc.VectorSubcoreMesh(core_axis_name="c", subcore_axis_name="s")

    @functools.partial(
        pl.kernel, mesh=mesh,
        out_type=jax.ShapeDtypeStruct((T, DIM), jnp.float32),
        scratch_types=[
            pltpu.VMEM((ct,), jnp.int32),
            pltpu.VMEM((ct,), jnp.int32),
            pltpu.VMEM((ct,), jnp.float32),
            pltpu.VMEM((ct,), jnp.float32),
            pltpu.VMEM((ct, DIM), jnp.float32),
            pltpu.VMEM((ct, DIM), jnp.float32),
            pltpu.VMEM((ct, DIM), jnp.float32),
            pltpu.SemaphoreType.DMA,
            pltpu.SemaphoreType.DMA,
        ],
    )
    def k(y_hbm, pf_hbm, wf_hbm, out_hbm,
          i0v, i1v, w0v, w1v, r0v, r1v, ov, sem0, sem1):
        wid = lax.axis_index("s") * _NC + lax.axis_index("c")
        ntok = T // _NW
        for j in range(ntok // ct):
            b = wid * ntok + j * ct
            pltpu.sync_copy(pf_hbm.at[pl.ds(b, ct)], i0v)
            pltpu.sync_copy(pf_hbm.at[pl.ds(T + b, ct)], i1v)
            pltpu.sync_copy(wf_hbm.at[pl.ds(b, ct)], w0v)
            pltpu.sync_copy(wf_hbm.at[pl.ds(T + b, ct)], w1v)
            c0 = pltpu.async_copy(y_hbm.at[i0v], r0v, sem0)
            c1 = pltpu.async_copy(y_hbm.at[i1v], r1v, sem1)
            c0.wait()
            c1.wait()
            w0a = w0v[...]
            w1a = w1v[...]
            wa = [w0a[tt] for tt in range(ct)]
            wb = [w1a[tt] for tt in range(ct)]

            def cbody(c, carry):
                for tt in range(ct):
                    sl = pl.ds(c * 16, 16)
                    ov[tt, sl] = r0v[tt, sl] * wa[tt] + r1v[tt, sl] * wb[tt]
                return carry

            lax.fori_loop(0, DIM // 16, cbody, 0)
            pltpu.sync_copy(ov, out_hbm.at[pl.ds(b, ct)])

    return k(y, pos_flat, w_flat)


@jax.jit
def kernel(hidden_states, Wg, Wu, Wd, Wr):
    b, s, d = hidden_states.shape
    x = hidden_states.reshape(T, d)

    pf, wf, te, tv, fs, ne, od = _router_sort(x, Wr)
    pos_flat = pf.reshape(2 * T)
    w_flat = wf.reshape(2 * T)
    xs = _dispatch_sc(x, pos_flat)
    y = _gmm(xs, Wg, Wu, Wd, te[0], tv[0], fs[0], ne[0], od[0])
    out = _combine_sc(y, pos_flat, w_flat)
    return out.reshape(b, s, d)


# speedup vs baseline: 2.5540x; 1.0402x over previous
import functools

import jax
import jax.numpy as jnp
from jax import lax
from jax.experimental import pallas as pl
from jax.experimental.pallas import tpu as pltpu
from jax.experimental.pallas import tpu_sc as plsc

DIM = 1024
HIDDEN = 2048
E = 8
T = 2048

TM3 = 256
NT3 = 24
M_PAD = NT3 * TM3
CS = 256
NTP = 32

_NC, _NS = 2, 16
_NW = _NC * _NS


def _router_body(x_ref, wrt_ref, pf_ref, wf_ref, te_ref, tv_ref,
                 fs_ref, ne_ref, od_ref):
    x = x_ref[...]
    xb = x.astype(jnp.bfloat16)
    logits = lax.dot_general(xb, wrt_ref[...].astype(jnp.bfloat16),
                             (((1,), (1,)), ((), ())),
                             preferred_element_type=jnp.float32)
    lanes = lax.broadcasted_iota(jnp.int32, (T, E), 1)
    m1 = jnp.max(logits, axis=1, keepdims=True)
    i1 = jnp.min(jnp.where(logits == m1, lanes, E), axis=1, keepdims=True)
    l2 = jnp.where(lanes == i1, -1e30, logits)
    m2 = jnp.max(l2, axis=1, keepdims=True)
    i2 = jnp.min(jnp.where(l2 == m2, lanes, E), axis=1, keepdims=True)
    r = jnp.exp(m2 - m1)
    den = 1.0 + r
    wf_ref[pl.ds(0, T), :] = 1.0 / den
    wf_ref[pl.ds(T, T), :] = r / den

    oh0 = (lanes == i1).astype(jnp.float32)
    oh1 = (lanes == i2).astype(jnp.float32)

    ii = lax.broadcasted_iota(jnp.int32, (CS, CS), 0)
    jj = lax.broadcasted_iota(jnp.int32, (CS, CS), 1)
    ltri = (jj < ii).astype(jnp.float32)
    off = jnp.zeros((1, E), jnp.float32)
    prefixes = []
    for oh in (oh0, oh1):
        parts = []
        for c in range(T // CS):
            blk = lax.slice(oh, (c * CS, 0), ((c + 1) * CS, E))
            ex = lax.dot(ltri, blk,
                         precision=lax.Precision.HIGHEST) + off
            off = off + jnp.sum(blk, axis=0, keepdims=True)
            parts.append(ex)
        prefixes.append(jnp.concatenate(parts, axis=0))
    counts = off
    padded = jnp.ceil(counts / TM3) * TM3

    acc = jnp.zeros((1, 1), jnp.float32)
    offp_parts = []
    for e in range(E):
        offp_parts.append(acc)
        acc = acc + lax.slice(padded, (0, e), (1, e + 1))
    offp = jnp.concatenate(offp_parts, axis=1)

    pos0 = jnp.sum(oh0 * (prefixes[0] + offp), axis=1, keepdims=True)
    pos1 = jnp.sum(oh1 * (prefixes[1] + offp), axis=1, keepdims=True)
    pf_ref[pl.ds(0, T), :] = pos0.astype(jnp.int32)
    pf_ref[pl.ds(T, T), :] = pos1.astype(jnp.int32)

    row0 = lax.broadcasted_iota(jnp.int32, (1, NTP), 1).astype(jnp.float32) * TM3
    active = (counts > 0)
    nxt_after = jnp.full((1, 1), -1.0, jnp.float32)
    nxts = [None] * E
    for e in range(E - 1, -1, -1):
        nxts[e] = nxt_after
        ae = lax.slice(active, (0, e), (1, e + 1))
        nxt_after = jnp.where(ae, float(e), nxt_after)
    nordacc = jnp.zeros((1, 1), jnp.float32)
    te = jnp.zeros((1, NTP), jnp.float32)
    fs = jnp.zeros((1, NTP), jnp.float32)
    ne = jnp.zeros((1, NTP), jnp.float32)
    od = jnp.zeros((1, NTP), jnp.float32)
    for e in range(E):
        oe = lax.slice(offp, (0, e), (1, e + 1))
        pe = lax.slice(padded, (0, e), (1, e + 1))
        inside = jnp.logical_and(row0 >= oe, row0 < oe + pe).astype(jnp.float32)
        te = te + e * inside
        fs = fs + jnp.where(row0 == oe, inside, 0.0)
        nv = jnp.where(nxts[e] < 0, float(e), nxts[e])
        ne = ne + nv * inside
        od = od + nordacc * inside
        ae = lax.slice(active, (0, e), (1, e + 1))
        nordacc = nordacc + ae.astype(jnp.float32)
    total = acc
    valid = row0 < total
    erow = lax.broadcasted_iota(jnp.int32, (1, E), 1).astype(jnp.float32)
    laste = jnp.max(jnp.where(counts > 0, erow, 0.0), axis=1, keepdims=True)
    te = jnp.where(valid, te, laste)
    te_ref[...] = te.astype(jnp.int32)
    tv_ref[...] = valid.astype(jnp.int32)
    fs_ref[...] = fs.astype(jnp.int32)
    ne_ref[...] = ne.astype(jnp.int32)
    od_ref[...] = (od - 2.0 * jnp.floor(od * 0.5)).astype(jnp.int32)


def _router_sort(x, wrt):
    outs = pl.pallas_call(
        _router_body,
        out_shape=[
            jax.ShapeDtypeStruct((2 * T, 1), jnp.int32),
            jax.ShapeDtypeStruct((2 * T, 1), jnp.float32),
            jax.ShapeDtypeStruct((1, NTP), jnp.int32),
            jax.ShapeDtypeStruct((1, NTP), jnp.int32),
            jax.ShapeDtypeStruct((1, NTP), jnp.int32),
            jax.ShapeDtypeStruct((1, NTP), jnp.int32),
            jax.ShapeDtypeStruct((1, NTP), jnp.int32),
        ],
    )(x, wrt)
    return outs


def _dispatch_sc(x, pos_flat):
    ch = 32
    nj = (2 * T // _NW) // ch
    mesh = plsc.VectorSubcoreMesh(core_axis_name="c", subcore_axis_name="s")

    @functools.partial(
        pl.kernel, mesh=mesh,
        out_type=jax.ShapeDtypeStruct((M_PAD, DIM), jnp.float32),
        scratch_types=[
            pltpu.VMEM((ch,), jnp.int32),
            pltpu.VMEM((ch,), jnp.int32),
            pltpu.VMEM((ch, DIM), jnp.float32),
            pltpu.VMEM((ch, DIM), jnp.float32),
            pltpu.SemaphoreType.DMA,
            pltpu.SemaphoreType.DMA,
        ],
    )
    def k(x_hbm, pos_hbm, xs_hbm, idx0, idx1, rows0, rows1, sem0, sem1):
        wid = lax.axis_index("s") * _NC + lax.axis_index("c")
        base = wid * (ch * nj)
        idx = (idx0, idx1)
        rows = (rows0, rows1)
        sems = (sem0, sem1)

        def load(j, bi):
            b = base + j * ch
            tok = lax.rem(b, T)
            pltpu.sync_copy(pos_hbm.at[pl.ds(b, ch)], idx[bi])
            pltpu.sync_copy(x_hbm.at[pl.ds(tok, ch)], rows[bi])

        load(0, 0)
        for j in range(nj):
            cur = j % 2
            pltpu.make_async_copy(rows[cur], xs_hbm.at[idx[cur]], sems[cur]).start()
            if j + 1 < nj:
                if j >= 1:
                    pltpu.make_async_copy(rows[1 - cur],
                                          xs_hbm.at[idx[1 - cur]],
                                          sems[1 - cur]).wait()
                load(j + 1, 1 - cur)
        pltpu.make_async_copy(rows[(nj - 2) % 2], xs_hbm.at[idx[(nj - 2) % 2]],
                              sems[(nj - 2) % 2]).wait()
        pltpu.make_async_copy(rows[(nj - 1) % 2], xs_hbm.at[idx[(nj - 1) % 2]],
                              sems[(nj - 1) % 2]).wait()

    return k(x, pos_flat)


def _gmm_body(te_r, tv_r, fs_r, ne_r, od_r,
              x_ref, wg_hbm, wu_hbm, wd_hbm, out_ref,
              wg0, wu0, wd0, wg1, wu1, wd1,
              sg0, su0, sd0, sg1, su1, sd1):
    m = pl.program_id(0)
    e = te_r[m]
    par = od_r[m]
    nxt = ne_r[m]
    first = fs_r[m] == 1

    def _issue(eidx, bg, bu, bd, s0, s1, s2):
        pltpu.make_async_copy(wg_hbm.at[eidx], bg, s0).start()
        pltpu.make_async_copy(wu_hbm.at[eidx], bu, s1).start()
        pltpu.make_async_copy(wd_hbm.at[eidx], bd, s2).start()

    def _wait(eidx, bg, bu, bd, s0, s1, s2):
        pltpu.make_async_copy(wg_hbm.at[eidx], bg, s0).wait()
        pltpu.make_async_copy(wu_hbm.at[eidx], bu, s1).wait()
        pltpu.make_async_copy(wd_hbm.at[eidx], bd, s2).wait()

    @pl.when(m == 0)
    def _():
        _issue(e, wg0, wu0, wd0, sg0, su0, sd0)

    @pl.when(jnp.logical_and(first, nxt != e))
    def _():
        @pl.when(par == 0)
        def _():
            _issue(nxt, wg1, wu1, wd1, sg1, su1, sd1)

        @pl.when(par == 1)
        def _():
            _issue(nxt, wg0, wu0, wd0, sg0, su0, sd0)

    @pl.when(first)
    def _():
        @pl.when(par == 0)
        def _():
            _wait(e, wg0, wu0, wd0, sg0, su0, sd0)

        @pl.when(par == 1)
        def _():
            _wait(e, wg1, wu1, wd1, sg1, su1, sd1)

    def _compute(bg, bu, bd):
        xv = x_ref[...]
        nt = (((1,), (1,)), ((), ()))
        g = lax.dot_general(xv, bg[...], nt, preferred_element_type=jnp.float32)
        u = lax.dot_general(xv, bu[...], nt, preferred_element_type=jnp.float32)
        act = (g * jax.nn.sigmoid(g)) * u
        out_ref[...] = lax.dot_general(act, bd[...], nt,
                                       preferred_element_type=jnp.float32)

    @pl.when(jnp.logical_and(tv_r[m] == 1, par == 0))
    def _():
        _compute(wg0, wu0, wd0)

    @pl.when(jnp.logical_and(tv_r[m] == 1, par == 1))
    def _():
        _compute(wg1, wu1, wd1)


def _gmm(xs, wgt, wut, wdt, te, tv, fs, ne, od):
    grid_spec = pltpu.PrefetchScalarGridSpec(
        num_scalar_prefetch=5,
        grid=(NT3,),
        in_specs=[
            pl.BlockSpec((TM3, DIM), lambda m, *_: (m, 0)),
            pl.BlockSpec(memory_space=pltpu.MemorySpace.HBM),
            pl.BlockSpec(memory_space=pltpu.MemorySpace.HBM),
            pl.BlockSpec(memory_space=pltpu.MemorySpace.HBM),
        ],
        out_specs=pl.BlockSpec((TM3, DIM), lambda m, *_: (m, 0)),
        scratch_shapes=[
            pltpu.VMEM((HIDDEN, DIM), jnp.float32),
            pltpu.VMEM((HIDDEN, DIM), jnp.float32),
            pltpu.VMEM((DIM, HIDDEN), jnp.float32),
            pltpu.VMEM((HIDDEN, DIM), jnp.float32),
            pltpu.VMEM((HIDDEN, DIM), jnp.float32),
            pltpu.VMEM((DIM, HIDDEN), jnp.float32),
        ] + [pltpu.SemaphoreType.DMA] * 6,
    )
    return pl.pallas_call(
        _gmm_body,
        grid_spec=grid_spec,
        out_shape=jax.ShapeDtypeStruct((M_PAD, DIM), jnp.float32),
        compiler_params=pltpu.CompilerParams(vmem_limit_bytes=100 * 1024 * 1024),
    )(te, tv, fs, ne, od, xs, wgt, wut, wdt)


def _combine_sc(y, pos_flat, w_flat):
    ct = 16
    nj = (T // _NW) // ct
    mesh = plsc.VectorSubcoreMesh(core_axis_name="c", subcore_axis_name="s")

    @functools.partial(
        pl.kernel, mesh=mesh,
        out_type=jax.ShapeDtypeStruct((T, DIM), jnp.float32),
        scratch_types=(
            [pltpu.VMEM((ct,), jnp.int32)] * 4
            + [pltpu.VMEM((ct,), jnp.float32)] * 4
            + [pltpu.VMEM((ct, DIM), jnp.float32)] * 6
            + [pltpu.SemaphoreType.DMA] * 6
        ),
    )
    def k(y_hbm, pf_hbm, wf_hbm, out_hbm,
          i0a, i0b, i1a, i1b, w0a_, w0b_, w1a_, w1b_,
          r0a, r0b, r1a, r1b, ova, ovb,
          s0a, s0b, s1a, s1b, soa, sob):
        wid = lax.axis_index("s") * _NC + lax.axis_index("c")
        ntok = T // _NW
        i0 = (i0a, i0b)
        i1 = (i1a, i1b)
        w0 = (w0a_, w0b_)
        w1 = (w1a_, w1b_)
        r0 = (r0a, r0b)
        r1 = (r1a, r1b)
        ov = (ova, ovb)
        s0 = (s0a, s0b)
        s1 = (s1a, s1b)
        so = (soa, sob)

        def issue(j, bi):
            b = wid * ntok + j * ct
            pltpu.sync_copy(pf_hbm.at[pl.ds(b, ct)], i0[bi])
            pltpu.sync_copy(pf_hbm.at[pl.ds(T + b, ct)], i1[bi])
            pltpu.sync_copy(wf_hbm.at[pl.ds(b, ct)], w0[bi])
            pltpu.sync_copy(wf_hbm.at[pl.ds(T + b, ct)], w1[bi])
            pltpu.make_async_copy(y_hbm.at[i0[bi]], r0[bi], s0[bi]).start()
            pltpu.make_async_copy(y_hbm.at[i1[bi]], r1[bi], s1[bi]).start()

        issue(0, 0)
        for j in range(nj):
            cur = j % 2
            b = wid * ntok + j * ct
            if j + 1 < nj:
                issue(j + 1, 1 - cur)
            pltpu.make_async_copy(y_hbm.at[i0[cur]], r0[cur], s0[cur]).wait()
            pltpu.make_async_copy(y_hbm.at[i1[cur]], r1[cur], s1[cur]).wait()
            if j >= 2:
                pltpu.make_async_copy(ov[cur], out_hbm.at[pl.ds(b, ct)],
                                      so[cur]).wait()
            wva = w0[cur][...]
            wvb = w1[cur][...]
            wa = [wva[tt] for tt in range(ct)]
            wb = [wvb[tt] for tt in range(ct)]
            r0c = r0[cur]
            r1c = r1[cur]
            ovc = ov[cur]

            def cbody(c, carry):
                for tt in range(ct):
                    sl = pl.ds(c * 16, 16)
                    ovc[tt, sl] = r0c[tt, sl] * wa[tt] + r1c[tt, sl] * wb[tt]
                return carry

            lax.fori_loop(0, DIM // 16, cbody, 0)
            pltpu.make_async_copy(ov[cur], out_hbm.at[pl.ds(b, ct)],
                                  so[cur]).start()
        for j in (nj - 2, nj - 1):
            b = wid * ntok + j * ct
            pltpu.make_async_copy(ov[j % 2], out_hbm.at[pl.ds(b, ct)],
                                  so[j % 2]).wait()

    return k(y, pos_flat, w_flat)


@jax.jit
def kernel(hidden_states, Wg, Wu, Wd, Wr):
    b, s, d = hidden_states.shape
    x = hidden_states.reshape(T, d)

    pf, wf, te, tv, fs, ne, od = _router_sort(x, Wr)
    pos_flat = pf.reshape(2 * T)
    w_flat = wf.reshape(2 * T)
    xs = _dispatch_sc(x, pos_flat)
    y = _gmm(xs, Wg, Wu, Wd, te[0], tv[0], fs[0], ne[0], od[0])
    out = _combine_sc(y, pos_flat, w_flat)
    return out.reshape(b, s, d)


# speedup vs baseline: 2.5560x; 1.0008x over previous
import functools

import jax
import jax.numpy as jnp
from jax import lax
from jax.experimental import pallas as pl
from jax.experimental.pallas import tpu as pltpu
from jax.experimental.pallas import tpu_sc as plsc

DIM = 1024
HIDDEN = 2048
E = 8
T = 2048

TM3 = 256
NT3 = 24
M_PAD = NT3 * TM3
CS = 256
NTP = 32

_NC, _NS = 2, 16
_NW = _NC * _NS


def _router_body(x_ref, wrt_ref, pf_ref, wf_ref, meta_ref):
    x = x_ref[...]
    xb = x.astype(jnp.bfloat16)
    logits = lax.dot_general(xb, wrt_ref[...].astype(jnp.bfloat16),
                             (((1,), (1,)), ((), ())),
                             preferred_element_type=jnp.float32)
    lanes = lax.broadcasted_iota(jnp.int32, (T, E), 1)
    m1 = jnp.max(logits, axis=1, keepdims=True)
    i1 = jnp.min(jnp.where(logits == m1, lanes, E), axis=1, keepdims=True)
    l2 = jnp.where(lanes == i1, -1e30, logits)
    m2 = jnp.max(l2, axis=1, keepdims=True)
    i2 = jnp.min(jnp.where(l2 == m2, lanes, E), axis=1, keepdims=True)
    r = jnp.exp(m2 - m1)
    den = 1.0 + r
    wf_ref[pl.ds(0, T), :] = 1.0 / den
    wf_ref[pl.ds(T, T), :] = r / den

    oh0 = (lanes == i1).astype(jnp.float32)
    oh1 = (lanes == i2).astype(jnp.float32)

    ii = lax.broadcasted_iota(jnp.int32, (CS, CS), 0)
    jj = lax.broadcasted_iota(jnp.int32, (CS, CS), 1)
    ltri = (jj < ii).astype(jnp.float32)
    off = jnp.zeros((1, E), jnp.float32)
    prefixes = []
    for oh in (oh0, oh1):
        parts = []
        for c in range(T // CS):
            blk = lax.slice(oh, (c * CS, 0), ((c + 1) * CS, E))
            ex = lax.dot(ltri, blk,
                         precision=lax.Precision.HIGHEST) + off
            off = off + jnp.sum(blk, axis=0, keepdims=True)
            parts.append(ex)
        prefixes.append(jnp.concatenate(parts, axis=0))
    counts = off
    padded = jnp.ceil(counts / TM3) * TM3

    acc = jnp.zeros((1, 1), jnp.float32)
    offp_parts = []
    for e in range(E):
        offp_parts.append(acc)
        acc = acc + lax.slice(padded, (0, e), (1, e + 1))
    offp = jnp.concatenate(offp_parts, axis=1)

    pos0 = jnp.sum(oh0 * (prefixes[0] + offp), axis=1, keepdims=True)
    pos1 = jnp.sum(oh1 * (prefixes[1] + offp), axis=1, keepdims=True)
    pf_ref[pl.ds(0, T), :] = pos0.astype(jnp.int32)
    pf_ref[pl.ds(T, T), :] = pos1.astype(jnp.int32)

    row0 = lax.broadcasted_iota(jnp.int32, (1, NTP), 1).astype(jnp.float32) * TM3
    active = (counts > 0)
    nxt_after = jnp.full((1, 1), -1.0, jnp.float32)
    nxts = [None] * E
    for e in range(E - 1, -1, -1):
        nxts[e] = nxt_after
        ae = lax.slice(active, (0, e), (1, e + 1))
        nxt_after = jnp.where(ae, float(e), nxt_after)
    nordacc = jnp.zeros((1, 1), jnp.float32)
    te = jnp.zeros((1, NTP), jnp.float32)
    fs = jnp.zeros((1, NTP), jnp.float32)
    ne = jnp.zeros((1, NTP), jnp.float32)
    od = jnp.zeros((1, NTP), jnp.float32)
    for e in range(E):
        oe = lax.slice(offp, (0, e), (1, e + 1))
        pe = lax.slice(padded, (0, e), (1, e + 1))
        inside = jnp.logical_and(row0 >= oe, row0 < oe + pe).astype(jnp.float32)
        te = te + e * inside
        fs = fs + jnp.where(row0 == oe, inside, 0.0)
        nv = jnp.where(nxts[e] < 0, float(e), nxts[e])
        ne = ne + nv * inside
        od = od + nordacc * inside
        ae = lax.slice(active, (0, e), (1, e + 1))
        nordacc = nordacc + ae.astype(jnp.float32)
    total = acc
    valid = row0 < total
    erow = lax.broadcasted_iota(jnp.int32, (1, E), 1).astype(jnp.float32)
    laste = jnp.max(jnp.where(counts > 0, erow, 0.0), axis=1, keepdims=True)
    te = jnp.where(valid, te, laste)
    odp = od - 2.0 * jnp.floor(od * 0.5)
    meta = jnp.concatenate(
        [te, valid.astype(jnp.float32), fs, ne, odp,
         jnp.zeros((3, NTP), jnp.float32)], axis=0)
    meta_ref[...] = meta.astype(jnp.int32)


def _router_sort(x, wrt):
    outs = pl.pallas_call(
        _router_body,
        out_shape=[
            jax.ShapeDtypeStruct((2 * T, 1), jnp.int32),
            jax.ShapeDtypeStruct((2 * T, 1), jnp.float32),
            jax.ShapeDtypeStruct((8, NTP), jnp.int32),
        ],
    )(x, wrt)
    return outs


def _dispatch_sc(x, pos_flat):
    ch = 32
    nj = (2 * T // _NW) // ch
    mesh = plsc.VectorSubcoreMesh(core_axis_name="c", subcore_axis_name="s")

    @functools.partial(
        pl.kernel, mesh=mesh,
        out_type=jax.ShapeDtypeStruct((M_PAD, DIM), jnp.float32),
        scratch_types=[
            pltpu.VMEM((ch,), jnp.int32),
            pltpu.VMEM((ch,), jnp.int32),
            pltpu.VMEM((ch, DIM), jnp.float32),
            pltpu.VMEM((ch, DIM), jnp.float32),
            pltpu.SemaphoreType.DMA,
            pltpu.SemaphoreType.DMA,
        ],
    )
    def k(x_hbm, pos_hbm, xs_hbm, idx0, idx1, rows0, rows1, sem0, sem1):
        wid = lax.axis_index("s") * _NC + lax.axis_index("c")
        base = wid * (ch * nj)
        idx = (idx0, idx1)
        rows = (rows0, rows1)
        sems = (sem0, sem1)

        def load(j, bi):
            b = base + j * ch
            tok = lax.rem(b, T)
            pltpu.sync_copy(pos_hbm.at[pl.ds(b, ch)], idx[bi])
            pltpu.sync_copy(x_hbm.at[pl.ds(tok, ch)], rows[bi])

        load(0, 0)
        for j in range(nj):
            cur = j % 2
            pltpu.make_async_copy(rows[cur], xs_hbm.at[idx[cur]], sems[cur]).start()
            if j + 1 < nj:
                if j >= 1:
                    pltpu.make_async_copy(rows[1 - cur],
                                          xs_hbm.at[idx[1 - cur]],
                                          sems[1 - cur]).wait()
                load(j + 1, 1 - cur)
        pltpu.make_async_copy(rows[(nj - 2) % 2], xs_hbm.at[idx[(nj - 2) % 2]],
                              sems[(nj - 2) % 2]).wait()
        pltpu.make_async_copy(rows[(nj - 1) % 2], xs_hbm.at[idx[(nj - 1) % 2]],
                              sems[(nj - 1) % 2]).wait()

    return k(x, pos_flat)


def _gmm_body(meta_r,
              x_ref, wg_hbm, wu_hbm, wd_hbm, out_ref,
              wg0, wu0, wd0, wg1, wu1, wd1,
              sg0, su0, sd0, sg1, su1, sd1):
    m = pl.program_id(0)
    e = meta_r[0, m]
    valid = meta_r[1, m] == 1
    first = meta_r[2, m] == 1
    nxt = meta_r[3, m]
    par = meta_r[4, m]

    def _issue(eidx, bg, bu, bd, s0, s1, s2):
        pltpu.make_async_copy(wg_hbm.at[eidx], bg, s0).start()
        pltpu.make_async_copy(wu_hbm.at[eidx], bu, s1).start()
        pltpu.make_async_copy(wd_hbm.at[eidx], bd, s2).start()

    def _wait(eidx, bg, bu, bd, s0, s1, s2):
        pltpu.make_async_copy(wg_hbm.at[eidx], bg, s0).wait()
        pltpu.make_async_copy(wu_hbm.at[eidx], bu, s1).wait()
        pltpu.make_async_copy(wd_hbm.at[eidx], bd, s2).wait()

    @pl.when(m == 0)
    def _():
        _issue(e, wg0, wu0, wd0, sg0, su0, sd0)

    @pl.when(jnp.logical_and(first, nxt != e))
    def _():
        @pl.when(par == 0)
        def _():
            _issue(nxt, wg1, wu1, wd1, sg1, su1, sd1)

        @pl.when(par == 1)
        def _():
            _issue(nxt, wg0, wu0, wd0, sg0, su0, sd0)

    @pl.when(first)
    def _():
        @pl.when(par == 0)
        def _():
            _wait(e, wg0, wu0, wd0, sg0, su0, sd0)

        @pl.when(par == 1)
        def _():
            _wait(e, wg1, wu1, wd1, sg1, su1, sd1)

    def _compute(bg, bu, bd):
        xv = x_ref[...]
        nt = (((1,), (1,)), ((), ()))
        g = lax.dot_general(xv, bg[...], nt, preferred_element_type=jnp.float32)
        u = lax.dot_general(xv, bu[...], nt, preferred_element_type=jnp.float32)
        act = (g * jax.nn.sigmoid(g)) * u
        out_ref[...] = lax.dot_general(act, bd[...], nt,
                                       preferred_element_type=jnp.float32)

    @pl.when(jnp.logical_and(valid, par == 0))
    def _():
        _compute(wg0, wu0, wd0)

    @pl.when(jnp.logical_and(valid, par == 1))
    def _():
        _compute(wg1, wu1, wd1)


def _gmm(xs, wgt, wut, wdt, meta):
    grid_spec = pltpu.PrefetchScalarGridSpec(
        num_scalar_prefetch=1,
        grid=(NT3,),
        in_specs=[
            pl.BlockSpec((TM3, DIM), lambda m, *_: (m, 0)),
            pl.BlockSpec(memory_space=pltpu.MemorySpace.HBM),
            pl.BlockSpec(memory_space=pltpu.MemorySpace.HBM),
            pl.BlockSpec(memory_space=pltpu.MemorySpace.HBM),
        ],
        out_specs=pl.BlockSpec((TM3, DIM), lambda m, *_: (m, 0)),
        scratch_shapes=[
            pltpu.VMEM((HIDDEN, DIM), jnp.float32),
            pltpu.VMEM((HIDDEN, DIM), jnp.float32),
            pltpu.VMEM((DIM, HIDDEN), jnp.float32),
            pltpu.VMEM((HIDDEN, DIM), jnp.float32),
            pltpu.VMEM((HIDDEN, DIM), jnp.float32),
            pltpu.VMEM((DIM, HIDDEN), jnp.float32),
        ] + [pltpu.SemaphoreType.DMA] * 6,
    )
    return pl.pallas_call(
        _gmm_body,
        grid_spec=grid_spec,
        out_shape=jax.ShapeDtypeStruct((M_PAD, DIM), jnp.float32),
        compiler_params=pltpu.CompilerParams(vmem_limit_bytes=100 * 1024 * 1024),
    )(meta, xs, wgt, wut, wdt)


def _combine_sc(y, pos_flat, w_flat):
    ct = 16
    nj = (T // _NW) // ct
    mesh = plsc.VectorSubcoreMesh(core_axis_name="c", subcore_axis_name="s")

    @functools.partial(
        pl.kernel, mesh=mesh,
        out_type=jax.ShapeDtypeStruct((T, DIM), jnp.float32),
        scratch_types=(
            [pltpu.VMEM((ct,), jnp.int32)] * 4
            + [pltpu.VMEM((ct,), jnp.float32)] * 4
            + [pltpu.VMEM((ct, DIM), jnp.float32)] * 6
            + [pltpu.SemaphoreType.DMA] * 6
        ),
    )
    def k(y_hbm, pf_hbm, wf_hbm, out_hbm,
          i0a, i0b, i1a, i1b, w0a_, w0b_, w1a_, w1b_,
          r0a, r0b, r1a, r1b, ova, ovb,
          s0a, s0b, s1a, s1b, soa, sob):
        wid = lax.axis_index("s") * _NC + lax.axis_index("c")
        ntok = T // _NW
        i0 = (i0a, i0b)
        i1 = (i1a, i1b)
        w0 = (w0a_, w0b_)
        w1 = (w1a_, w1b_)
        r0 = (r0a, r0b)
        r1 = (r1a, r1b)
        ov = (ova, ovb)
        s0 = (s0a, s0b)
        s1 = (s1a, s1b)
        so = (soa, sob)

        def issue(j, bi):
            b = wid * ntok + j * ct
            pltpu.sync_copy(pf_hbm.at[pl.ds(b, ct)], i0[bi])
            pltpu.sync_copy(pf_hbm.at[pl.ds(T + b, ct)], i1[bi])
            pltpu.sync_copy(wf_hbm.at[pl.ds(b, ct)], w0[bi])
            pltpu.sync_copy(wf_hbm.at[pl.ds(T + b, ct)], w1[bi])
            pltpu.make_async_copy(y_hbm.at[i0[bi]], r0[bi], s0[bi]).start()
            pltpu.make_async_copy(y_hbm.at[i1[bi]], r1[bi], s1[bi]).start()

        issue(0, 0)
        for j in range(nj):
            cur = j % 2
            b = wid * ntok + j * ct
            if j + 1 < nj:
                issue(j + 1, 1 - cur)
            pltpu.make_async_copy(y_hbm.at[i0[cur]], r0[cur], s0[cur]).wait()
            pltpu.make_async_copy(y_hbm.at[i1[cur]], r1[cur], s1[cur]).wait()
            if j >= 2:
                pltpu.make_async_copy(ov[cur], out_hbm.at[pl.ds(b, ct)],
                                      so[cur]).wait()
            wva = w0[cur][...]
            wvb = w1[cur][...]
            wa = [wva[tt] for tt in range(ct)]
            wb = [wvb[tt] for tt in range(ct)]
            r0c = r0[cur]
            r1c = r1[cur]
            ovc = ov[cur]

            def cbody(c, carry):
                for tt in range(ct):
                    sl = pl.ds(c * 16, 16)
                    ovc[tt, sl] = r0c[tt, sl] * wa[tt] + r1c[tt, sl] * wb[tt]
                return carry

            lax.fori_loop(0, DIM // 16, cbody, 0)
            pltpu.make_async_copy(ov[cur], out_hbm.at[pl.ds(b, ct)],
                                  so[cur]).start()
        for j in (nj - 2, nj - 1):
            b = wid * ntok + j * ct
            pltpu.make_async_copy(ov[j % 2], out_hbm.at[pl.ds(b, ct)],
                                  so[j % 2]).wait()

    return k(y, pos_flat, w_flat)


@jax.jit
def kernel(hidden_states, Wg, Wu, Wd, Wr):
    b, s, d = hidden_states.shape
    x = hidden_states.reshape(T, d)

    pf, wf, meta = _router_sort(x, Wr)
    pos_flat = pf.reshape(2 * T)
    w_flat = wf.reshape(2 * T)
    xs = _dispatch_sc(x, pos_flat)
    y = _gmm(xs, Wg, Wu, Wd, meta)
    out = _combine_sc(y, pos_flat, w_flat)
    return out.reshape(b, s, d)
